# baseline (speedup 1.0000x reference)
import functools

import jax
import jax.numpy as jnp
from jax import lax
from jax.experimental import pallas as pl
from jax.experimental.pallas import tpu as pltpu
from jax.experimental.pallas import tpu_sc as plsc

B, T = 64, 128
D = 256
TS_LEN = 16
BT = B * T
R = BT * TS_LEN
VOCAB = 1024

DISC_BASE = 1024
SPECIAL_BASE = 2048
EMB_ROWS = 2064
DW = D // 2

NC, NS = 2, 16
NW = NC * NS
ROWS_PER_W = R // NW
CHUNK = 64
NCH = ROWS_PER_W // CHUNK
BS = 12
NB = B - BS
VT = 1024


def _prep_body(co_ref, do_ref, ca_ref, sp_ref, pobs_ref, pact_ref, pts_ref,
               pos_ref, pos2_ref, idx_ref):
    def tok(x):
        u = (jnp.tanh(x) + 1.0) * 0.5
        return jnp.clip(jnp.floor(u * VOCAB).astype(jnp.int32), 0, VOCAB - 1)

    tco = tok(co_ref[...])
    tca = tok(ca_ref[...])
    dd = do_ref[...] + DISC_BASE
    sp = jnp.full((BT, 1), SPECIAL_BASE, jnp.int32)
    idx_ref[...] = jnp.concatenate([tco, dd, sp, tca], axis=1)

    pos_slot = jnp.concatenate([pobs_ref[...], pact_ref[...]], axis=0)
    pos_comb = pts_ref[...][:, None, :] + pos_slot[None, :, :]
    pos_ref[...] = pos_comb.reshape(T * TS_LEN, D)
    is_sp = lax.broadcasted_iota(jnp.int32, (TS_LEN, 1), 0) == 12
    pos_slot2 = pos_slot + jnp.where(is_sp, sp_ref[0:1, :], 0.0)
    pos2 = pts_ref[...][:, None, :] + pos_slot2[None, :, :]
    pos2_ref[...] = pos2.reshape(T * TS_LEN, D)


_prep = pl.pallas_call(
    _prep_body,
    out_shape=(
        jax.ShapeDtypeStruct((T * TS_LEN, D), jnp.float32),
        jax.ShapeDtypeStruct((T * TS_LEN, D), jnp.float32),
        jax.ShapeDtypeStruct((BT, TS_LEN), jnp.int32),
    ),
)


def _tcg_body(idx_ref, tbl_ref, pos_ref, out_ref):
    t = idx_ref[...]
    acc = pos_ref[...]
    for vt in range(2048 // VT):
        vio = lax.broadcasted_iota(jnp.int32, (T * TS_LEN, VT), 1) + vt * VT
        oh = jnp.where(vio == t, 1.0, 0.0).astype(jnp.bfloat16)
        acc = acc + jax.lax.dot_general(
            oh, tbl_ref[pl.ds(vt * VT, VT), :],
            (((1,), (0,)), ((), ())),
            preferred_element_type=jnp.float32)
    out_ref[...] = acc


_tcg = pl.pallas_call(
    _tcg_body,
    grid=(NB,),
    in_specs=[
        pl.BlockSpec((T * TS_LEN, 1), lambda i: (i, 0)),
        pl.BlockSpec((2048, D), lambda i: (0, 0)),
        pl.BlockSpec((T * TS_LEN, D), lambda i: (0, 0)),
    ],
    out_specs=pl.BlockSpec((T * TS_LEN, D), lambda i: (i, 0)),
    out_shape=jax.ShapeDtypeStruct((NB * T * TS_LEN, D), jnp.float32),
    compiler_params=pltpu.CompilerParams(dimension_semantics=("parallel",)),
)


def _sc_body(emb_hbm, idx_hbm, pos_hbm, out_hbm, idxbuf, posbuf,
             gbuf0, gbuf1, gbuf2, gbuf3, obuf0, obuf1,
             gsem0, gsem1, gsem2, gsem3, ssem0, ssem1):
    w = lax.axis_index("s") * NC + lax.axis_index("c")

    pltpu.sync_copy(pos_hbm.at[pl.ds(w * CHUNK, CHUNK)], posbuf)
    pltpu.sync_copy(idx_hbm.at[pl.ds(0, BS), w], idxbuf)

    gbufs = (gbuf0, gbuf1, gbuf2, gbuf3)
    gsems = (gsem0, gsem1, gsem2, gsem3)
    obufs = (obuf0, obuf1)
    ssems = (ssem0, ssem1)
    HIMASK = jnp.int32(-65536)

    def out_off(j):
        return pl.multiple_of((w + 32 * j) * CHUNK, CHUNK)

    def gather(j, p):
        pltpu.async_copy(emb_hbm.at[idxbuf.at[j]], gbufs[p], gsems[p])

    def wait_gather(p):
        pltpu.make_async_copy(emb_hbm.at[pl.ds(0, CHUNK)], gbufs[p],
                              gsems[p]).wait()

    def scatter(j, o):
        pltpu.async_copy(obufs[o], out_hbm.at[pl.ds(out_off(j), CHUNK)],
                         ssems[o])

    def wait_scatter(o):
        pltpu.make_async_copy(obufs[o], out_hbm.at[pl.ds(0, CHUNK)],
                              ssems[o]).wait()

    def unpack_add(p, o):
        g = gbufs[p]
        ob = obufs[o]

        def row(gi, carry):
            for l in range(DW // 16):
                wvec = g[gi, pl.ds(l * 16, 16)]
                lo = plsc.bitcast(wvec << 16, jnp.float32)
                hi = plsc.bitcast(wvec & HIMASK, jnp.float32)
                sl = pl.ds(l * 32, 16)
                sh = pl.ds(l * 32 + 16, 16)
                ob[gi, sl] = lo + posbuf[gi, sl]
                ob[gi, sh] = hi + posbuf[gi, sh]
            return carry

        lax.fori_loop(0, CHUNK, row, 0, unroll=2)

    gather(0, 0)
    gather(1, 1)
    gather(2, 2)
    gather(3, 3)
    wait_gather(0)
    unpack_add(0, 0)
    scatter(0, 0)
    gather(4, 0)
    wait_gather(1)
    unpack_add(1, 1)
    scatter(1, 1)

    def body(j, carry):
        jn = jnp.minimum(j + 3, BS - 1)

        def step(p, f):
            gather(jn, f)
            o = p % 2
            wait_scatter(o)
            wait_gather(p)
            unpack_add(p, o)
            scatter(j, o)

        for p in range(4):
            @pl.when(j % 4 == p)
            def _(p=p):
                step(p, (p + 3) % 4)

        return carry

    lax.fori_loop(2, BS, body, 0)

    wait_scatter(0)
    wait_scatter(1)
    wait_gather(0)
    wait_gather(1)
    wait_gather(2)


@functools.lru_cache(maxsize=1)
def _sc_gather():
    return functools.partial(
        pl.kernel,
        out_type=jax.ShapeDtypeStruct((BS * 2048, D), jnp.float32),
        mesh=plsc.VectorSubcoreMesh(core_axis_name="c", subcore_axis_name="s",
                                    num_cores=NC, num_subcores=NS),
        compiler_params=pltpu.CompilerParams(needs_layout_passes=False),
        scratch_types=[
            pltpu.VMEM((BS, CHUNK), jnp.int32),
            pltpu.VMEM((CHUNK, D), jnp.float32),
            pltpu.VMEM((CHUNK, DW), jnp.int32),
            pltpu.VMEM((CHUNK, DW), jnp.int32),
            pltpu.VMEM((CHUNK, DW), jnp.int32),
            pltpu.VMEM((CHUNK, DW), jnp.int32),
            pltpu.VMEM((CHUNK, D), jnp.float32),
            pltpu.VMEM((CHUNK, D), jnp.float32),
        ] + [pltpu.SemaphoreType.DMA] * 6,
    )(_sc_body)


def kernel(continuous_obs, discrete_obs, continuous_act, cont_table, disc_table,
           special_table, pos_obs_table, pos_act_table, pos_ts_table):
    co2 = continuous_obs.reshape(BT, 8)
    do2 = discrete_obs.reshape(BT, 4)
    ca2 = continuous_act.reshape(BT, 3)
    pos, pos2, idx = _prep(co2, do2, ca2, special_table, pos_obs_table,
                           pos_act_table, pos_ts_table)

    emb = jnp.concatenate([cont_table, disc_table, special_table], axis=0)
    embp = emb.astype(jnp.bfloat16).reshape(EMB_ROWS, DW // 16, 2, 16)
    embp = embp.transpose(0, 1, 3, 2)
    emb_pk = lax.bitcast_convert_type(embp, jnp.int32).reshape(EMB_ROWS, DW)

    sc_out = _sc_gather()(emb_pk, idx[:BS * T].reshape(BS, 32, CHUNK), pos)
    tc_out = _tcg(idx[BS * T:].reshape(NB * T * TS_LEN, 1),
                  emb[:2048].astype(jnp.bfloat16), pos2)
    out = jnp.concatenate([sc_out.reshape(BS, T * TS_LEN, D),
                           tc_out.reshape(NB, T * TS_LEN, D)], axis=0)
    return out.reshape(B, T * TS_LEN, D)


# speedup vs baseline: 1.0143x; 1.0143x over previous
import functools

import jax
import jax.numpy as jnp
from jax import lax
from jax.experimental import pallas as pl
from jax.experimental.pallas import tpu as pltpu
from jax.experimental.pallas import tpu_sc as plsc

B, T = 64, 128
D = 256
TS_LEN = 16
BT = B * T
R = BT * TS_LEN
VOCAB = 1024

DISC_BASE = 1024
SPECIAL_BASE = 2048
EMB_ROWS = 2064
DW = D // 2

NC, NS = 2, 16
NW = NC * NS
ROWS_PER_W = R // NW
CHUNK = 64
NCH = ROWS_PER_W // CHUNK
BS = 8
NB = B - BS
VT = 1024


def _prep_body(co_ref, do_ref, ca_ref, sp_ref, pobs_ref, pact_ref, pts_ref,
               pos_ref, pos2_ref, idx_ref):
    def tok(x):
        u = (jnp.tanh(x) + 1.0) * 0.5
        return jnp.clip(jnp.floor(u * VOCAB).astype(jnp.int32), 0, VOCAB - 1)

    tco = tok(co_ref[...])
    tca = tok(ca_ref[...])
    dd = do_ref[...] + DISC_BASE
    sp = jnp.full((BT, 1), SPECIAL_BASE, jnp.int32)
    idx_ref[...] = jnp.concatenate([tco, dd, sp, tca], axis=1)

    pos_slot = jnp.concatenate([pobs_ref[...], pact_ref[...]], axis=0)
    pos_comb = pts_ref[...][:, None, :] + pos_slot[None, :, :]
    pos_ref[...] = pos_comb.reshape(T * TS_LEN, D)
    is_sp = lax.broadcasted_iota(jnp.int32, (TS_LEN, 1), 0) == 12
    pos_slot2 = pos_slot + jnp.where(is_sp, sp_ref[0:1, :], 0.0)
    pos2 = pts_ref[...][:, None, :] + pos_slot2[None, :, :]
    pos2_ref[...] = pos2.reshape(T * TS_LEN, D)


_prep = pl.pallas_call(
    _prep_body,
    out_shape=(
        jax.ShapeDtypeStruct((T * TS_LEN, D), jnp.float32),
        jax.ShapeDtypeStruct((T * TS_LEN, D), jnp.float32),
        jax.ShapeDtypeStruct((BT, TS_LEN), jnp.int32),
    ),
)


def _tcg_body(idx_ref, tbl_ref, pos_ref, out_ref):
    t = idx_ref[...]
    acc = pos_ref[...]
    for vt in range(2048 // VT):
        vio = lax.broadcasted_iota(jnp.int32, (T * TS_LEN, VT), 1) + vt * VT
        oh = jnp.where(vio == t, 1.0, 0.0).astype(jnp.bfloat16)
        acc = acc + jax.lax.dot_general(
            oh, tbl_ref[pl.ds(vt * VT, VT), :],
            (((1,), (0,)), ((), ())),
            preferred_element_type=jnp.float32)
    out_ref[...] = acc


_tcg = pl.pallas_call(
    _tcg_body,
    grid=(NB,),
    in_specs=[
        pl.BlockSpec((T * TS_LEN, 1), lambda i: (i, 0)),
        pl.BlockSpec((2048, D), lambda i: (0, 0)),
        pl.BlockSpec((T * TS_LEN, D), lambda i: (0, 0)),
    ],
    out_specs=pl.BlockSpec((T * TS_LEN, D), lambda i: (i, 0)),
    out_shape=jax.ShapeDtypeStruct((NB * T * TS_LEN, D), jnp.float32),
    compiler_params=pltpu.CompilerParams(dimension_semantics=("parallel",)),
)


def _sc_body(emb_hbm, idx_hbm, pos_hbm, out_hbm, idxbuf, posbuf,
             gbuf0, gbuf1, gbuf2, gbuf3, obuf0, obuf1,
             gsem0, gsem1, gsem2, gsem3, ssem0, ssem1):
    w = lax.axis_index("s") * NC + lax.axis_index("c")

    pltpu.sync_copy(pos_hbm.at[pl.ds(w * CHUNK, CHUNK)], posbuf)
    pltpu.sync_copy(idx_hbm.at[pl.ds(0, BS), w], idxbuf)

    gbufs = (gbuf0, gbuf1, gbuf2, gbuf3)
    gsems = (gsem0, gsem1, gsem2, gsem3)
    obufs = (obuf0, obuf1)
    ssems = (ssem0, ssem1)
    HIMASK = jnp.int32(-65536)

    def out_off(j):
        return pl.multiple_of((w + 32 * j) * CHUNK, CHUNK)

    def gather(j, p):
        pltpu.async_copy(emb_hbm.at[idxbuf.at[j]], gbufs[p], gsems[p])

    def wait_gather(p):
        pltpu.make_async_copy(emb_hbm.at[pl.ds(0, CHUNK)], gbufs[p],
                              gsems[p]).wait()

    def scatter(j, o):
        pltpu.async_copy(obufs[o], out_hbm.at[pl.ds(out_off(j), CHUNK)],
                         ssems[o])

    def wait_scatter(o):
        pltpu.make_async_copy(obufs[o], out_hbm.at[pl.ds(0, CHUNK)],
                              ssems[o]).wait()

    def unpack_add(p, o):
        g = gbufs[p]
        ob = obufs[o]

        def row(gi, carry):
            for l in range(DW // 16):
                wvec = g[gi, pl.ds(l * 16, 16)]
                lo = plsc.bitcast(wvec << 16, jnp.float32)
                hi = plsc.bitcast(wvec & HIMASK, jnp.float32)
                sl = pl.ds(l * 32, 16)
                sh = pl.ds(l * 32 + 16, 16)
                ob[gi, sl] = lo + posbuf[gi, sl]
                ob[gi, sh] = hi + posbuf[gi, sh]
            return carry

        lax.fori_loop(0, CHUNK, row, 0, unroll=2)

    gather(0, 0)
    gather(1, 1)
    gather(2, 2)
    gather(3, 3)
    wait_gather(0)
    unpack_add(0, 0)
    scatter(0, 0)
    gather(4, 0)
    wait_gather(1)
    unpack_add(1, 1)
    scatter(1, 1)

    def body(j, carry):
        jn = jnp.minimum(j + 3, BS - 1)

        def step(p, f):
            gather(jn, f)
            o = p % 2
            wait_scatter(o)
            wait_gather(p)
            unpack_add(p, o)
            scatter(j, o)

        for p in range(4):
            @pl.when(j % 4 == p)
            def _(p=p):
                step(p, (p + 3) % 4)

        return carry

    lax.fori_loop(2, BS, body, 0)

    wait_scatter(0)
    wait_scatter(1)
    wait_gather(0)
    wait_gather(1)
    wait_gather(2)


@functools.lru_cache(maxsize=1)
def _sc_gather():
    return functools.partial(
        pl.kernel,
        out_type=jax.ShapeDtypeStruct((BS * 2048, D), jnp.float32),
        mesh=plsc.VectorSubcoreMesh(core_axis_name="c", subcore_axis_name="s",
                                    num_cores=NC, num_subcores=NS),
        compiler_params=pltpu.CompilerParams(needs_layout_passes=False),
        scratch_types=[
            pltpu.VMEM((BS, CHUNK), jnp.int32),
            pltpu.VMEM((CHUNK, D), jnp.float32),
            pltpu.VMEM((CHUNK, DW), jnp.int32),
            pltpu.VMEM((CHUNK, DW), jnp.int32),
            pltpu.VMEM((CHUNK, DW), jnp.int32),
            pltpu.VMEM((CHUNK, DW), jnp.int32),
            pltpu.VMEM((CHUNK, D), jnp.float32),
            pltpu.VMEM((CHUNK, D), jnp.float32),
        ] + [pltpu.SemaphoreType.DMA] * 6,
    )(_sc_body)


def kernel(continuous_obs, discrete_obs, continuous_act, cont_table, disc_table,
           special_table, pos_obs_table, pos_act_table, pos_ts_table):
    co2 = continuous_obs.reshape(BT, 8)
    do2 = discrete_obs.reshape(BT, 4)
    ca2 = continuous_act.reshape(BT, 3)
    pos, pos2, idx = _prep(co2, do2, ca2, special_table, pos_obs_table,
                           pos_act_table, pos_ts_table)

    emb = jnp.concatenate([cont_table, disc_table, special_table], axis=0)
    embp = emb.astype(jnp.bfloat16).reshape(EMB_ROWS, DW // 16, 2, 16)
    embp = embp.transpose(0, 1, 3, 2)
    emb_pk = lax.bitcast_convert_type(embp, jnp.int32).reshape(EMB_ROWS, DW)

    sc_out = _sc_gather()(emb_pk, idx[:BS * T].reshape(BS, 32, CHUNK), pos)
    tc_out = _tcg(idx[BS * T:].reshape(NB * T * TS_LEN, 1),
                  emb[:2048].astype(jnp.bfloat16), pos2)
    out = jnp.concatenate([sc_out.reshape(BS, T * TS_LEN, D),
                           tc_out.reshape(NB, T * TS_LEN, D)], axis=0)
    return out.reshape(B, T * TS_LEN, D)
